# baseline (speedup 1.0000x reference)
import functools

import jax
import jax.numpy as jnp
from jax.experimental import pallas as pl
from jax.experimental.pallas import tpu as pltpu

_B, _N, _M, _P = 2, 10000, 10000, 128
_K, _KC, _FEAT = 32, 4, 128
_SIGMA = 0.01
_C = 10112
_QB = 256
_BIG = 1.0e9
_HIGH = jax.lax.Precision.HIGHEST


def _dot(a, b):
    return jax.lax.dot_general(a, b, (((1,), (0,)), ((), ())),
                               precision=_HIGH,
                               preferred_element_type=jnp.float32)


def _knn1_kernel(noisyT_ref, sidx_ref, w1_ref, b1_ref, w2_ref, b2_ref,
                 q_ref, feat_ref, fx_ref, fy_ref, fz_ref, d_ref):
    px = noisyT_ref[0, 0:1, :]
    py = noisyT_ref[0, 1:2, :]
    pz = noisyT_ref[0, 2:3, :]
    iota = jax.lax.broadcasted_iota(jnp.int32, (1, _C), 1)
    sidx = sidx_ref[:, :]
    sel = iota == sidx
    zeros = jnp.zeros((_P, _C), jnp.float32)
    qx = jnp.sum(jnp.where(sel, px, zeros), axis=1, keepdims=True)
    qy = jnp.sum(jnp.where(sel, py, zeros), axis=1, keepdims=True)
    qz = jnp.sum(jnp.where(sel, pz, zeros), axis=1, keepdims=True)
    q3 = jnp.concatenate([qx, qy, qz], axis=1)
    q_ref[0, :, :] = q3

    h = jnp.maximum(_dot(q3, w1_ref[:, :]) + b1_ref[0:1, :], 0.0)
    feat_ref[0, :, :] = _dot(h, w2_ref[:, :]) + b2_ref[0:1, :]

    pn = px * px + py * py + pz * pz
    d_ref[:, :] = pn - 2.0 * _dot(q3, noisyT_ref[0, 0:3, :])

    kiota = jax.lax.broadcasted_iota(jnp.int32, (1, _K), 1)
    big_i = jnp.int32(2**30)

    def body(k, carry):
        fx, fy, fz = carry
        d = d_ref[:, :]
        m = jnp.min(d, axis=1, keepdims=True)
        hit = d == m
        idx = jnp.min(jnp.where(hit, iota, big_i), axis=1, keepdims=True)
        one = iota == idx
        cx = jnp.sum(jnp.where(one, px, zeros), axis=1, keepdims=True)
        cy = jnp.sum(jnp.where(one, py, zeros), axis=1, keepdims=True)
        cz = jnp.sum(jnp.where(one, pz, zeros), axis=1, keepdims=True)
        d_ref[:, :] = jnp.where(one, _BIG, d)
        colk = kiota == k
        fx = jnp.where(colk, cx, fx)
        fy = jnp.where(colk, cy, fy)
        fz = jnp.where(colk, cz, fz)
        return fx, fy, fz

    init = (jnp.zeros((_P, _K), jnp.float32),
            jnp.zeros((_P, _K), jnp.float32),
            jnp.zeros((_P, _K), jnp.float32))
    fx, fy, fz = jax.lax.fori_loop(0, _K, body, init)
    fx_ref[0, :, :] = fx
    fy_ref[0, :, :] = fy
    fz_ref[0, :, :] = fz


def _knn2_kernel(f4_ref, q4_ref, featrep_ref, cleanT_ref,
                 ws1a_ref, ws1b_ref, bs1_ref, ws2_ref, bs2_ref,
                 out_ref):
    b = pl.program_id(0)
    j = pl.program_id(1)

    @pl.when(jnp.logical_and(b == 0, j == 0))
    def _():
        out_ref[0, 0] = 0.0

    f4 = f4_ref[0, :, :]
    q4 = q4_ref[0, :, :]
    disp = (f4 - q4)[:, 0:3]
    h = _dot(disp, ws1a_ref[:, :]) + _dot(featrep_ref[0, :, :], ws1b_ref[:, :])
    h = jnp.maximum(h + bs1_ref[0:1, :], 0.0)
    estim = _dot(h, ws2_ref[:, :]) + bs2_ref[0:1, :]

    px = cleanT_ref[0, 0:1, :]
    py = cleanT_ref[0, 1:2, :]
    pz = cleanT_ref[0, 2:3, :]
    fx = f4[:, 0:1]
    fy = f4[:, 1:2]
    fz = f4[:, 2:3]
    pn = px * px + py * py + pz * pz
    d = pn - 2.0 * _dot(f4[:, 0:3], cleanT_ref[0, 0:3, :])
    m1 = jnp.min(d, axis=1, keepdims=True)
    m2 = jnp.min(jnp.where(d > m1, d, _BIG), axis=1, keepdims=True)
    m3 = jnp.min(jnp.where(d > m2, d, _BIG), axis=1, keepdims=True)
    m4 = jnp.min(jnp.where(d > m3, d, _BIG), axis=1, keepdims=True)
    sel = d <= m4
    zeros = jnp.zeros((_QB, _C), jnp.float32)
    cnt = jnp.sum(jnp.where(sel, jnp.ones((_QB, _C), jnp.float32), zeros),
                  axis=1, keepdims=True)
    csx = jnp.sum(jnp.where(sel, px, zeros), axis=1, keepdims=True)
    csy = jnp.sum(jnp.where(sel, py, zeros), axis=1, keepdims=True)
    csz = jnp.sum(jnp.where(sel, pz, zeros), axis=1, keepdims=True)
    gx = csx / cnt - fx
    gy = csy / cnt - fy
    gz = csz / cnt - fz
    ex = estim[:, 0:1]
    ey = estim[:, 1:2]
    ez = estim[:, 2:3]
    err = (ex - gx) ** 2 + (ey - gy) ** 2 + (ez - gz) ** 2
    scale = 0.5 * (1.0 / _SIGMA) / (_B * _P * _K)
    out_ref[0, 0] += jnp.sum(err) * scale


def kernel(noisy_pc, clean_pc, sampled_idx, W1, b1, W2, b2, Ws1, bs1, Ws2, bs2):
    f32 = jnp.float32
    noisyT = jnp.transpose(noisy_pc.astype(f32), (0, 2, 1))
    noisyT = jnp.pad(noisyT, ((0, 0), (0, 5), (0, _C - _N)),
                     constant_values=_BIG)
    cleanT = jnp.transpose(clean_pc.astype(f32), (0, 2, 1))
    cleanT = jnp.pad(cleanT, ((0, 0), (0, 5), (0, _C - _M)),
                     constant_values=_BIG)
    sidx2d = sampled_idx.astype(jnp.int32).reshape(_P, 1)

    q, feat, fx, fy, fz = pl.pallas_call(
        _knn1_kernel,
        grid=(_B,),
        in_specs=[
            pl.BlockSpec((1, 8, _C), lambda b: (b, 0, 0)),
            pl.BlockSpec((_P, 1), lambda b: (0, 0)),
            pl.BlockSpec((3, _FEAT), lambda b: (0, 0)),
            pl.BlockSpec((1, _FEAT), lambda b: (0, 0)),
            pl.BlockSpec((_FEAT, _FEAT), lambda b: (0, 0)),
            pl.BlockSpec((1, _FEAT), lambda b: (0, 0)),
        ],
        out_specs=[
            pl.BlockSpec((1, _P, 3), lambda b: (b, 0, 0)),
            pl.BlockSpec((1, _P, _FEAT), lambda b: (b, 0, 0)),
            pl.BlockSpec((1, _P, _K), lambda b: (b, 0, 0)),
            pl.BlockSpec((1, _P, _K), lambda b: (b, 0, 0)),
            pl.BlockSpec((1, _P, _K), lambda b: (b, 0, 0)),
        ],
        out_shape=[
            jax.ShapeDtypeStruct((_B, _P, 3), f32),
            jax.ShapeDtypeStruct((_B, _P, _FEAT), f32),
            jax.ShapeDtypeStruct((_B, _P, _K), f32),
            jax.ShapeDtypeStruct((_B, _P, _K), f32),
            jax.ShapeDtypeStruct((_B, _P, _K), f32),
        ],
        scratch_shapes=[pltpu.VMEM((_P, _C), f32)],
    )(noisyT, sidx2d, W1.astype(f32), b1.reshape(1, _FEAT).astype(f32),
      W2.astype(f32), b2.reshape(1, _FEAT).astype(f32))

    f4 = jnp.stack([fx, fy, fz, jnp.zeros_like(fx)], axis=-1)
    f4 = f4.reshape(_B, _P * _K, 4)
    q4 = jnp.pad(q, ((0, 0), (0, 0), (0, 1)))
    q4 = jnp.repeat(q4, _K, axis=1)
    featrep = jnp.repeat(feat, _K, axis=1)

    nblk = (_P * _K) // _QB
    loss = pl.pallas_call(
        _knn2_kernel,
        grid=(_B, nblk),
        in_specs=[
            pl.BlockSpec((1, _QB, 4), lambda b, j: (b, j, 0)),
            pl.BlockSpec((1, _QB, 4), lambda b, j: (b, j, 0)),
            pl.BlockSpec((1, _QB, _FEAT), lambda b, j: (b, j, 0)),
            pl.BlockSpec((1, 8, _C), lambda b, j: (b, 0, 0)),
            pl.BlockSpec((3, _FEAT), lambda b, j: (0, 0)),
            pl.BlockSpec((_FEAT, _FEAT), lambda b, j: (0, 0)),
            pl.BlockSpec((1, _FEAT), lambda b, j: (0, 0)),
            pl.BlockSpec((_FEAT, 3), lambda b, j: (0, 0)),
            pl.BlockSpec((1, 3), lambda b, j: (0, 0)),
        ],
        out_specs=pl.BlockSpec((1, 1), lambda b, j: (0, 0),
                               memory_space=pltpu.SMEM),
        out_shape=jax.ShapeDtypeStruct((1, 1), f32),
    )(f4, q4, featrep, cleanT,
      Ws1[0:3, :].astype(f32), Ws1[3:, :].astype(f32),
      bs1.reshape(1, _FEAT).astype(f32),
      Ws2.astype(f32), bs2.reshape(1, 3).astype(f32))

    return loss[0, 0]


# speedup vs baseline: 1.2316x; 1.2316x over previous
import functools

import jax
import jax.numpy as jnp
from jax.experimental import pallas as pl
from jax.experimental.pallas import tpu as pltpu

_B, _N, _M, _P = 2, 10000, 10000, 128
_K, _KC, _FEAT = 32, 4, 128
_SIGMA = 0.01
_C = 10112
_QB = 256
_BIG = 1.0e9
_HIGH = jax.lax.Precision.HIGHEST


def _dot(a, b):
    return jax.lax.dot_general(a, b, (((1,), (0,)), ((), ())),
                               precision=_HIGH,
                               preferred_element_type=jnp.float32)


def _knn1_kernel(noisyT_ref, sidx_ref, w1_ref, b1_ref, w2_ref, b2_ref,
                 q_ref, feat_ref, fx_ref, fy_ref, fz_ref, d_ref):
    px = noisyT_ref[0, 0:1, :]
    py = noisyT_ref[0, 1:2, :]
    pz = noisyT_ref[0, 2:3, :]
    iota = jax.lax.broadcasted_iota(jnp.int32, (1, _C), 1)
    sidx = sidx_ref[:, :]
    sel = iota == sidx
    zeros = jnp.zeros((_P, _C), jnp.float32)
    qx = jnp.sum(jnp.where(sel, px, zeros), axis=1, keepdims=True)
    qy = jnp.sum(jnp.where(sel, py, zeros), axis=1, keepdims=True)
    qz = jnp.sum(jnp.where(sel, pz, zeros), axis=1, keepdims=True)
    q3 = jnp.concatenate([qx, qy, qz], axis=1)
    q_ref[0, :, :] = q3

    h = jnp.maximum(_dot(q3, w1_ref[:, :]) + b1_ref[0:1, :], 0.0)
    feat_ref[0, :, :] = _dot(h, w2_ref[:, :]) + b2_ref[0:1, :]

    d_ref[:, :] = (qx - px) ** 2 + (qy - py) ** 2 + (qz - pz) ** 2

    kiota = jax.lax.broadcasted_iota(jnp.int32, (1, _K), 1)
    big_i = jnp.int32(2**30)

    def body(k, carry):
        fx, fy, fz = carry
        d = d_ref[:, :]
        m = jnp.min(d, axis=1, keepdims=True)
        hit = d == m
        idx = jnp.min(jnp.where(hit, iota, big_i), axis=1, keepdims=True)
        one = iota == idx
        cx = jnp.sum(jnp.where(one, px, zeros), axis=1, keepdims=True)
        cy = jnp.sum(jnp.where(one, py, zeros), axis=1, keepdims=True)
        cz = jnp.sum(jnp.where(one, pz, zeros), axis=1, keepdims=True)
        d_ref[:, :] = jnp.where(one, _BIG, d)
        colk = kiota == k
        fx = jnp.where(colk, cx, fx)
        fy = jnp.where(colk, cy, fy)
        fz = jnp.where(colk, cz, fz)
        return fx, fy, fz

    init = (jnp.zeros((_P, _K), jnp.float32),
            jnp.zeros((_P, _K), jnp.float32),
            jnp.zeros((_P, _K), jnp.float32))
    fx, fy, fz = jax.lax.fori_loop(0, _K, body, init)
    fx_ref[0, :, :] = fx
    fy_ref[0, :, :] = fy
    fz_ref[0, :, :] = fz


def _knn2_kernel(f4_ref, q4_ref, featrep_ref, cleanT_ref,
                 ws1a_ref, ws1b_ref, bs1_ref, ws2_ref, bs2_ref,
                 out_ref):
    b = pl.program_id(0)
    j = pl.program_id(1)

    @pl.when(jnp.logical_and(b == 0, j == 0))
    def _():
        out_ref[0, 0] = 0.0

    f4 = f4_ref[0, :, :]
    q4 = q4_ref[0, :, :]
    disp = (f4 - q4)[:, 0:3]
    h = _dot(disp, ws1a_ref[:, :]) + _dot(featrep_ref[0, :, :], ws1b_ref[:, :])
    h = jnp.maximum(h + bs1_ref[0:1, :], 0.0)
    estim = _dot(h, ws2_ref[:, :]) + bs2_ref[0:1, :]

    px = cleanT_ref[0, 0:1, :]
    py = cleanT_ref[0, 1:2, :]
    pz = cleanT_ref[0, 2:3, :]
    fx = f4[:, 0:1]
    fy = f4[:, 1:2]
    fz = f4[:, 2:3]
    d = (fx - px) ** 2 + (fy - py) ** 2 + (fz - pz) ** 2
    m1 = jnp.min(d, axis=1, keepdims=True)
    m2 = jnp.min(jnp.where(d > m1, d, _BIG), axis=1, keepdims=True)
    m3 = jnp.min(jnp.where(d > m2, d, _BIG), axis=1, keepdims=True)
    m4 = jnp.min(jnp.where(d > m3, d, _BIG), axis=1, keepdims=True)
    sel = d <= m4
    zeros = jnp.zeros((_QB, _C), jnp.float32)
    cnt = jnp.sum(jnp.where(sel, jnp.ones((_QB, _C), jnp.float32), zeros),
                  axis=1, keepdims=True)
    csx = jnp.sum(jnp.where(sel, px, zeros), axis=1, keepdims=True)
    csy = jnp.sum(jnp.where(sel, py, zeros), axis=1, keepdims=True)
    csz = jnp.sum(jnp.where(sel, pz, zeros), axis=1, keepdims=True)
    gx = csx / cnt - fx
    gy = csy / cnt - fy
    gz = csz / cnt - fz
    ex = estim[:, 0:1]
    ey = estim[:, 1:2]
    ez = estim[:, 2:3]
    err = (ex - gx) ** 2 + (ey - gy) ** 2 + (ez - gz) ** 2
    scale = 0.5 * (1.0 / _SIGMA) / (_B * _P * _K)
    out_ref[0, 0] += jnp.sum(err) * scale


def kernel(noisy_pc, clean_pc, sampled_idx, W1, b1, W2, b2, Ws1, bs1, Ws2, bs2):
    f32 = jnp.float32
    noisyT = jnp.transpose(noisy_pc.astype(f32), (0, 2, 1))
    noisyT = jnp.pad(noisyT, ((0, 0), (0, 5), (0, _C - _N)),
                     constant_values=_BIG)
    cleanT = jnp.transpose(clean_pc.astype(f32), (0, 2, 1))
    cleanT = jnp.pad(cleanT, ((0, 0), (0, 5), (0, _C - _M)),
                     constant_values=_BIG)
    sidx2d = sampled_idx.astype(jnp.int32).reshape(_P, 1)

    q, feat, fx, fy, fz = pl.pallas_call(
        _knn1_kernel,
        grid=(_B,),
        in_specs=[
            pl.BlockSpec((1, 8, _C), lambda b: (b, 0, 0)),
            pl.BlockSpec((_P, 1), lambda b: (0, 0)),
            pl.BlockSpec((3, _FEAT), lambda b: (0, 0)),
            pl.BlockSpec((1, _FEAT), lambda b: (0, 0)),
            pl.BlockSpec((_FEAT, _FEAT), lambda b: (0, 0)),
            pl.BlockSpec((1, _FEAT), lambda b: (0, 0)),
        ],
        out_specs=[
            pl.BlockSpec((1, _P, 3), lambda b: (b, 0, 0)),
            pl.BlockSpec((1, _P, _FEAT), lambda b: (b, 0, 0)),
            pl.BlockSpec((1, _P, _K), lambda b: (b, 0, 0)),
            pl.BlockSpec((1, _P, _K), lambda b: (b, 0, 0)),
            pl.BlockSpec((1, _P, _K), lambda b: (b, 0, 0)),
        ],
        out_shape=[
            jax.ShapeDtypeStruct((_B, _P, 3), f32),
            jax.ShapeDtypeStruct((_B, _P, _FEAT), f32),
            jax.ShapeDtypeStruct((_B, _P, _K), f32),
            jax.ShapeDtypeStruct((_B, _P, _K), f32),
            jax.ShapeDtypeStruct((_B, _P, _K), f32),
        ],
        scratch_shapes=[pltpu.VMEM((_P, _C), f32)],
    )(noisyT, sidx2d, W1.astype(f32), b1.reshape(1, _FEAT).astype(f32),
      W2.astype(f32), b2.reshape(1, _FEAT).astype(f32))

    f4 = jnp.stack([fx, fy, fz, jnp.zeros_like(fx)], axis=-1)
    f4 = f4.reshape(_B, _P * _K, 4)
    q4 = jnp.pad(q, ((0, 0), (0, 0), (0, 1)))
    q4 = jnp.repeat(q4, _K, axis=1)
    featrep = jnp.repeat(feat, _K, axis=1)

    nblk = (_P * _K) // _QB
    loss = pl.pallas_call(
        _knn2_kernel,
        grid=(_B, nblk),
        in_specs=[
            pl.BlockSpec((1, _QB, 4), lambda b, j: (b, j, 0)),
            pl.BlockSpec((1, _QB, 4), lambda b, j: (b, j, 0)),
            pl.BlockSpec((1, _QB, _FEAT), lambda b, j: (b, j, 0)),
            pl.BlockSpec((1, 8, _C), lambda b, j: (b, 0, 0)),
            pl.BlockSpec((3, _FEAT), lambda b, j: (0, 0)),
            pl.BlockSpec((_FEAT, _FEAT), lambda b, j: (0, 0)),
            pl.BlockSpec((1, _FEAT), lambda b, j: (0, 0)),
            pl.BlockSpec((_FEAT, 3), lambda b, j: (0, 0)),
            pl.BlockSpec((1, 3), lambda b, j: (0, 0)),
        ],
        out_specs=pl.BlockSpec((1, 1), lambda b, j: (0, 0),
                               memory_space=pltpu.SMEM),
        out_shape=jax.ShapeDtypeStruct((1, 1), f32),
    )(f4, q4, featrep, cleanT,
      Ws1[0:3, :].astype(f32), Ws1[3:, :].astype(f32),
      bs1.reshape(1, _FEAT).astype(f32),
      Ws2.astype(f32), bs2.reshape(1, 3).astype(f32))

    return loss[0, 0]


# speedup vs baseline: 1.4663x; 1.1906x over previous
import functools

import jax
import jax.numpy as jnp
from jax import lax
from jax.experimental import pallas as pl
from jax.experimental.pallas import tpu as pltpu
from jax.experimental.pallas import tpu_sc as plsc

_B, _N, _M, _P = 2, 10000, 10000, 128
_K, _KC, _FEAT = 32, 4, 128
_SIGMA = 0.01
_C = 10112
_QB = 256
_BIG = 1.0e9
_HIGH = jax.lax.Precision.HIGHEST


def _dot(a, b):
    return jax.lax.dot_general(a, b, (((1,), (0,)), ((), ())),
                               precision=_HIGH,
                               preferred_element_type=jnp.float32)


def _knn1_kernel(noisyT_ref, sidx_ref, w1_ref, b1_ref, w2_ref, b2_ref,
                 q_ref, feat_ref, idx_ref, d_ref):
    px = noisyT_ref[0, 0:1, :]
    py = noisyT_ref[0, 1:2, :]
    pz = noisyT_ref[0, 2:3, :]
    iota = jax.lax.broadcasted_iota(jnp.int32, (1, _C), 1)
    sidx = sidx_ref[:, :]
    sel = iota == sidx
    zeros = jnp.zeros((_P, _C), jnp.float32)
    qx = jnp.sum(jnp.where(sel, px, zeros), axis=1, keepdims=True)
    qy = jnp.sum(jnp.where(sel, py, zeros), axis=1, keepdims=True)
    qz = jnp.sum(jnp.where(sel, pz, zeros), axis=1, keepdims=True)
    q3 = jnp.concatenate([qx, qy, qz], axis=1)
    q_ref[0, :, :] = q3

    h = jnp.maximum(_dot(q3, w1_ref[:, :]) + b1_ref[0:1, :], 0.0)
    feat_ref[0, :, :] = _dot(h, w2_ref[:, :]) + b2_ref[0:1, :]

    d_ref[:, :] = (qx - px) ** 2 + (qy - py) ** 2 + (qz - pz) ** 2

    kiota = jax.lax.broadcasted_iota(jnp.int32, (1, _K), 1)
    big_i = jnp.int32(2**30)
    off = pl.program_id(0) * _C

    def body(k, idxs):
        d = d_ref[:, :]
        m = jnp.min(d, axis=1, keepdims=True)
        hit = d == m
        idx = jnp.min(jnp.where(hit, iota, big_i), axis=1, keepdims=True)
        d_ref[:, :] = jnp.where(iota == idx, _BIG, d)
        colk = kiota == k
        return jnp.where(colk, idx + off, idxs)

    idxs = jax.lax.fori_loop(0, _K, body,
                             jnp.zeros((_P, _K), jnp.int32))
    idx_ref[0, :, :] = idxs


def _knn2_kernel(f4_ref, q4_ref, featrep_ref, cleanT_ref,
                 ws1a_ref, ws1b_ref, bs1_ref, ws2_ref, bs2_ref,
                 out_ref):
    b = pl.program_id(0)
    j = pl.program_id(1)

    @pl.when(jnp.logical_and(b == 0, j == 0))
    def _():
        out_ref[0, 0] = 0.0

    f4 = f4_ref[0, :, :]
    q4 = q4_ref[0, :, :]
    disp = (f4 - q4)[:, 0:3]
    h = _dot(disp, ws1a_ref[:, :]) + _dot(featrep_ref[0, :, :], ws1b_ref[:, :])
    h = jnp.maximum(h + bs1_ref[0:1, :], 0.0)
    estim = _dot(h, ws2_ref[:, :]) + bs2_ref[0:1, :]

    px = cleanT_ref[0, 0:1, :]
    py = cleanT_ref[0, 1:2, :]
    pz = cleanT_ref[0, 2:3, :]
    fx = f4[:, 0:1]
    fy = f4[:, 1:2]
    fz = f4[:, 2:3]
    d = (fx - px) ** 2 + (fy - py) ** 2 + (fz - pz) ** 2
    m1 = jnp.min(d, axis=1, keepdims=True)
    m2 = jnp.min(jnp.where(d > m1, d, _BIG), axis=1, keepdims=True)
    m3 = jnp.min(jnp.where(d > m2, d, _BIG), axis=1, keepdims=True)
    m4 = jnp.min(jnp.where(d > m3, d, _BIG), axis=1, keepdims=True)
    sel = d <= m4
    zeros = jnp.zeros((_QB, _C), jnp.float32)
    cnt = jnp.sum(jnp.where(sel, jnp.ones((_QB, _C), jnp.float32), zeros),
                  axis=1, keepdims=True)
    csx = jnp.sum(jnp.where(sel, px, zeros), axis=1, keepdims=True)
    csy = jnp.sum(jnp.where(sel, py, zeros), axis=1, keepdims=True)
    csz = jnp.sum(jnp.where(sel, pz, zeros), axis=1, keepdims=True)
    gx = csx / cnt - fx
    gy = csy / cnt - fy
    gz = csz / cnt - fz
    ex = estim[:, 0:1]
    ey = estim[:, 1:2]
    ez = estim[:, 2:3]
    err = (ex - gx) ** 2 + (ey - gy) ** 2 + (ez - gz) ** 2
    scale = 0.5 * (1.0 / _SIGMA) / (_B * _P * _K)
    out_ref[0, 0] += jnp.sum(err) * scale


_NROWS = _B * _P * _K
_ROWD = 128


def _sc_gather(idx_flat, table):
    info = plsc.get_sparse_core_info()
    nw = info.num_cores * info.num_subcores
    per_w = _NROWS // nw
    mesh = plsc.VectorSubcoreMesh(core_axis_name="c", subcore_axis_name="s")

    @functools.partial(
        pl.kernel, mesh=mesh,
        out_type=jax.ShapeDtypeStruct((_NROWS, _ROWD), jnp.float32),
        scratch_types=[
            pltpu.VMEM((per_w,), jnp.int32),
            pltpu.VMEM((per_w, _ROWD), jnp.float32),
            pltpu.SemaphoreType.DMA,
        ],
    )
    def gk(idx_hbm, table_hbm, out_hbm, idx_v, rows_v, sem):
        wid = lax.axis_index("s") * info.num_cores + lax.axis_index("c")
        base = wid * per_w
        pltpu.sync_copy(idx_hbm.at[pl.ds(base, per_w)], idx_v)
        pltpu.async_copy(table_hbm.at[idx_v], rows_v, sem).wait()
        pltpu.sync_copy(rows_v, out_hbm.at[pl.ds(base, per_w)])

    return gk(idx_flat, table)


def kernel(noisy_pc, clean_pc, sampled_idx, W1, b1, W2, b2, Ws1, bs1, Ws2, bs2):
    f32 = jnp.float32
    noisyT = jnp.transpose(noisy_pc.astype(f32), (0, 2, 1))
    noisyT = jnp.pad(noisyT, ((0, 0), (0, 5), (0, _C - _N)),
                     constant_values=_BIG)
    cleanT = jnp.transpose(clean_pc.astype(f32), (0, 2, 1))
    cleanT = jnp.pad(cleanT, ((0, 0), (0, 5), (0, _C - _M)),
                     constant_values=_BIG)
    sidx2d = sampled_idx.astype(jnp.int32).reshape(_P, 1)

    q, feat, idx1g = pl.pallas_call(
        _knn1_kernel,
        grid=(_B,),
        in_specs=[
            pl.BlockSpec((1, 8, _C), lambda b: (b, 0, 0)),
            pl.BlockSpec((_P, 1), lambda b: (0, 0)),
            pl.BlockSpec((3, _FEAT), lambda b: (0, 0)),
            pl.BlockSpec((1, _FEAT), lambda b: (0, 0)),
            pl.BlockSpec((_FEAT, _FEAT), lambda b: (0, 0)),
            pl.BlockSpec((1, _FEAT), lambda b: (0, 0)),
        ],
        out_specs=[
            pl.BlockSpec((1, _P, 3), lambda b: (b, 0, 0)),
            pl.BlockSpec((1, _P, _FEAT), lambda b: (b, 0, 0)),
            pl.BlockSpec((1, _P, _K), lambda b: (b, 0, 0)),
        ],
        out_shape=[
            jax.ShapeDtypeStruct((_B, _P, 3), f32),
            jax.ShapeDtypeStruct((_B, _P, _FEAT), f32),
            jax.ShapeDtypeStruct((_B, _P, _K), jnp.int32),
        ],
        scratch_shapes=[pltpu.VMEM((_P, _C), f32)],
    )(noisyT, sidx2d, W1.astype(f32), b1.reshape(1, _FEAT).astype(f32),
      W2.astype(f32), b2.reshape(1, _FEAT).astype(f32))

    table = jnp.pad(noisy_pc.astype(f32),
                    ((0, 0), (0, _C - _N), (0, _ROWD - 3))
                    ).reshape(_B * _C, _ROWD)
    frows = _sc_gather(idx1g.reshape(_NROWS), table)
    f4 = frows[:, 0:4].reshape(_B, _P * _K, 4)
    q4 = jnp.pad(q, ((0, 0), (0, 0), (0, 1)))
    q4 = jnp.repeat(q4, _K, axis=1)
    featrep = jnp.repeat(feat, _K, axis=1)

    nblk = (_P * _K) // _QB
    loss = pl.pallas_call(
        _knn2_kernel,
        grid=(_B, nblk),
        in_specs=[
            pl.BlockSpec((1, _QB, 4), lambda b, j: (b, j, 0)),
            pl.BlockSpec((1, _QB, 4), lambda b, j: (b, j, 0)),
            pl.BlockSpec((1, _QB, _FEAT), lambda b, j: (b, j, 0)),
            pl.BlockSpec((1, 8, _C), lambda b, j: (b, 0, 0)),
            pl.BlockSpec((3, _FEAT), lambda b, j: (0, 0)),
            pl.BlockSpec((_FEAT, _FEAT), lambda b, j: (0, 0)),
            pl.BlockSpec((1, _FEAT), lambda b, j: (0, 0)),
            pl.BlockSpec((_FEAT, 3), lambda b, j: (0, 0)),
            pl.BlockSpec((1, 3), lambda b, j: (0, 0)),
        ],
        out_specs=pl.BlockSpec((1, 1), lambda b, j: (0, 0),
                               memory_space=pltpu.SMEM),
        out_shape=jax.ShapeDtypeStruct((1, 1), f32),
    )(f4, q4, featrep, cleanT,
      Ws1[0:3, :].astype(f32), Ws1[3:, :].astype(f32),
      bs1.reshape(1, _FEAT).astype(f32),
      Ws2.astype(f32), bs2.reshape(1, 3).astype(f32))

    return loss[0, 0]


# speedup vs baseline: 1.5177x; 1.0350x over previous
import functools

import jax
import jax.numpy as jnp
from jax import lax
from jax.experimental import pallas as pl
from jax.experimental.pallas import tpu as pltpu
from jax.experimental.pallas import tpu_sc as plsc

_B, _N, _M, _P = 2, 10000, 10000, 128
_K, _KC, _FEAT = 32, 4, 128
_SIGMA = 0.01
_C = 10112
_QB = 512
_BIG = 1.0e9
_HIGH = jax.lax.Precision.HIGHEST


def _dot(a, b):
    return jax.lax.dot_general(a, b, (((1,), (0,)), ((), ())),
                               precision=_HIGH,
                               preferred_element_type=jnp.float32)


def _knn1_kernel(noisyT_ref, sidx_ref, w1_ref, b1_ref, w2_ref, b2_ref,
                 q_ref, feat_ref, idx_ref, d_ref):
    px = noisyT_ref[0, 0:1, :]
    py = noisyT_ref[0, 1:2, :]
    pz = noisyT_ref[0, 2:3, :]
    iota = jax.lax.broadcasted_iota(jnp.int32, (1, _C), 1)
    sidx = sidx_ref[:, :]
    sel = iota == sidx
    zeros = jnp.zeros((_P, _C), jnp.float32)
    qx = jnp.sum(jnp.where(sel, px, zeros), axis=1, keepdims=True)
    qy = jnp.sum(jnp.where(sel, py, zeros), axis=1, keepdims=True)
    qz = jnp.sum(jnp.where(sel, pz, zeros), axis=1, keepdims=True)
    q3 = jnp.concatenate([qx, qy, qz], axis=1)
    q_ref[0, :, :] = q3

    h = jnp.maximum(_dot(q3, w1_ref[:, :]) + b1_ref[0:1, :], 0.0)
    feat_ref[0, :, :] = _dot(h, w2_ref[:, :]) + b2_ref[0:1, :]

    d_ref[:, :] = (qx - px) ** 2 + (qy - py) ** 2 + (qz - pz) ** 2

    kiota = jax.lax.broadcasted_iota(jnp.int32, (1, _K), 1)
    big_i = jnp.int32(2**30)
    off = pl.program_id(0) * _C

    def body(k, idxs):
        d = d_ref[:, :]
        m = jnp.min(d, axis=1, keepdims=True)
        hit = d == m
        idx = jnp.min(jnp.where(hit, iota, big_i), axis=1, keepdims=True)
        d_ref[:, :] = jnp.where(iota == idx, _BIG, d)
        colk = kiota == k
        return jnp.where(colk, idx + off, idxs)

    idxs = jax.lax.fori_loop(0, _K, body,
                             jnp.zeros((_P, _K), jnp.int32))
    idx_ref[0, :, :] = idxs


def _knn2_kernel(f4_ref, q4_ref, featrep_ref, cleanT_ref,
                 ws1a_ref, ws1b_ref, bs1_ref, ws2_ref, bs2_ref,
                 out_ref):
    b = pl.program_id(0)
    j = pl.program_id(1)

    @pl.when(jnp.logical_and(b == 0, j == 0))
    def _():
        out_ref[0, 0] = 0.0

    f4 = f4_ref[0, :, :]
    q4 = q4_ref[0, :, :]
    disp = (f4 - q4)[:, 0:3]
    h = _dot(disp, ws1a_ref[:, :]) + _dot(featrep_ref[0, :, :], ws1b_ref[:, :])
    h = jnp.maximum(h + bs1_ref[0:1, :], 0.0)
    estim = _dot(h, ws2_ref[:, :]) + bs2_ref[0:1, :]

    px = cleanT_ref[0, 0:1, :]
    py = cleanT_ref[0, 1:2, :]
    pz = cleanT_ref[0, 2:3, :]
    fx = f4[:, 0:1]
    fy = f4[:, 1:2]
    fz = f4[:, 2:3]
    d = (fx - px) ** 2 + (fy - py) ** 2 + (fz - pz) ** 2
    m1 = jnp.min(d, axis=1, keepdims=True)
    m2 = jnp.min(jnp.where(d > m1, d, _BIG), axis=1, keepdims=True)
    m3 = jnp.min(jnp.where(d > m2, d, _BIG), axis=1, keepdims=True)
    m4 = jnp.min(jnp.where(d > m3, d, _BIG), axis=1, keepdims=True)
    sel = d <= m4
    zeros = jnp.zeros((_QB, _C), jnp.float32)
    cnt = jnp.sum(jnp.where(sel, jnp.ones((_QB, _C), jnp.float32), zeros),
                  axis=1, keepdims=True)
    csx = jnp.sum(jnp.where(sel, px, zeros), axis=1, keepdims=True)
    csy = jnp.sum(jnp.where(sel, py, zeros), axis=1, keepdims=True)
    csz = jnp.sum(jnp.where(sel, pz, zeros), axis=1, keepdims=True)
    gx = csx / cnt - fx
    gy = csy / cnt - fy
    gz = csz / cnt - fz
    ex = estim[:, 0:1]
    ey = estim[:, 1:2]
    ez = estim[:, 2:3]
    err = (ex - gx) ** 2 + (ey - gy) ** 2 + (ez - gz) ** 2
    scale = 0.5 * (1.0 / _SIGMA) / (_B * _P * _K)
    out_ref[0, 0] += jnp.sum(err) * scale


_NROWS = _B * _P * _K
_ROWD = 128


def _sc_gather(idx_flat, table):
    info = plsc.get_sparse_core_info()
    nw = info.num_cores * info.num_subcores
    per_w = _NROWS // nw
    mesh = plsc.VectorSubcoreMesh(core_axis_name="c", subcore_axis_name="s")

    @functools.partial(
        pl.kernel, mesh=mesh,
        out_type=jax.ShapeDtypeStruct((_NROWS, _ROWD), jnp.float32),
        scratch_types=[
            pltpu.VMEM((per_w,), jnp.int32),
            pltpu.VMEM((per_w, _ROWD), jnp.float32),
            pltpu.SemaphoreType.DMA,
        ],
    )
    def gk(idx_hbm, table_hbm, out_hbm, idx_v, rows_v, sem):
        wid = lax.axis_index("s") * info.num_cores + lax.axis_index("c")
        base = wid * per_w
        pltpu.sync_copy(idx_hbm.at[pl.ds(base, per_w)], idx_v)
        pltpu.async_copy(table_hbm.at[idx_v], rows_v, sem).wait()
        pltpu.sync_copy(rows_v, out_hbm.at[pl.ds(base, per_w)])

    return gk(idx_flat, table)


def kernel(noisy_pc, clean_pc, sampled_idx, W1, b1, W2, b2, Ws1, bs1, Ws2, bs2):
    f32 = jnp.float32
    noisyT = jnp.transpose(noisy_pc.astype(f32), (0, 2, 1))
    noisyT = jnp.pad(noisyT, ((0, 0), (0, 5), (0, _C - _N)),
                     constant_values=_BIG)
    cleanT = jnp.transpose(clean_pc.astype(f32), (0, 2, 1))
    cleanT = jnp.pad(cleanT, ((0, 0), (0, 5), (0, _C - _M)),
                     constant_values=_BIG)
    sidx2d = sampled_idx.astype(jnp.int32).reshape(_P, 1)

    q, feat, idx1g = pl.pallas_call(
        _knn1_kernel,
        grid=(_B,),
        in_specs=[
            pl.BlockSpec((1, 8, _C), lambda b: (b, 0, 0)),
            pl.BlockSpec((_P, 1), lambda b: (0, 0)),
            pl.BlockSpec((3, _FEAT), lambda b: (0, 0)),
            pl.BlockSpec((1, _FEAT), lambda b: (0, 0)),
            pl.BlockSpec((_FEAT, _FEAT), lambda b: (0, 0)),
            pl.BlockSpec((1, _FEAT), lambda b: (0, 0)),
        ],
        out_specs=[
            pl.BlockSpec((1, _P, 3), lambda b: (b, 0, 0)),
            pl.BlockSpec((1, _P, _FEAT), lambda b: (b, 0, 0)),
            pl.BlockSpec((1, _P, _K), lambda b: (b, 0, 0)),
        ],
        out_shape=[
            jax.ShapeDtypeStruct((_B, _P, 3), f32),
            jax.ShapeDtypeStruct((_B, _P, _FEAT), f32),
            jax.ShapeDtypeStruct((_B, _P, _K), jnp.int32),
        ],
        scratch_shapes=[pltpu.VMEM((_P, _C), f32)],
    )(noisyT, sidx2d, W1.astype(f32), b1.reshape(1, _FEAT).astype(f32),
      W2.astype(f32), b2.reshape(1, _FEAT).astype(f32))

    table = jnp.pad(noisy_pc.astype(f32),
                    ((0, 0), (0, _C - _N), (0, _ROWD - 3))
                    ).reshape(_B * _C, _ROWD)
    frows = _sc_gather(idx1g.reshape(_NROWS), table)
    f4 = frows[:, 0:4].reshape(_B, _P * _K, 4)
    q4 = jnp.pad(q, ((0, 0), (0, 0), (0, 1)))
    q4 = jnp.repeat(q4, _K, axis=1)
    featrep = jnp.repeat(feat, _K, axis=1)

    nblk = (_P * _K) // _QB
    loss = pl.pallas_call(
        _knn2_kernel,
        grid=(_B, nblk),
        in_specs=[
            pl.BlockSpec((1, _QB, 4), lambda b, j: (b, j, 0)),
            pl.BlockSpec((1, _QB, 4), lambda b, j: (b, j, 0)),
            pl.BlockSpec((1, _QB, _FEAT), lambda b, j: (b, j, 0)),
            pl.BlockSpec((1, 8, _C), lambda b, j: (b, 0, 0)),
            pl.BlockSpec((3, _FEAT), lambda b, j: (0, 0)),
            pl.BlockSpec((_FEAT, _FEAT), lambda b, j: (0, 0)),
            pl.BlockSpec((1, _FEAT), lambda b, j: (0, 0)),
            pl.BlockSpec((_FEAT, 3), lambda b, j: (0, 0)),
            pl.BlockSpec((1, 3), lambda b, j: (0, 0)),
        ],
        out_specs=pl.BlockSpec((1, 1), lambda b, j: (0, 0),
                               memory_space=pltpu.SMEM),
        out_shape=jax.ShapeDtypeStruct((1, 1), f32),
    )(f4, q4, featrep, cleanT,
      Ws1[0:3, :].astype(f32), Ws1[3:, :].astype(f32),
      bs1.reshape(1, _FEAT).astype(f32),
      Ws2.astype(f32), bs2.reshape(1, 3).astype(f32))

    return loss[0, 0]
